# baseline (speedup 1.0000x reference)
import dataclasses
import functools

import jax
import jax.numpy as jnp
from jax import lax
from jax.experimental import pallas as pl
from jax.experimental.pallas import tpu as pltpu
from jax.experimental.pallas import tpu_sc as plsc

N = 10000
E = 320000
D = 128

NC = 2
NS = 16
NW = NC * NS

CHUNK = 128
K = 80
E_PAD = NW * K * CHUNK
PAD = E_PAD - E
G = 8
N_PAD = 10112
N_TRASH = N_PAD - N
ROWS_PER_SUB = N_PAD // NS

_MESH = plsc.VectorSubcoreMesh(
    core_axis_name="c", subcore_axis_name="s", num_cores=NC, num_subcores=NS
)

_CP = pltpu.CompilerParams()
if "needs_layout_passes" in pltpu.CompilerParams.__dataclass_fields__:
    _CP = dataclasses.replace(_CP, needs_layout_passes=False)


@functools.partial(
    pl.kernel,
    out_type=jax.ShapeDtypeStruct((2, NW, N_PAD), jnp.float32),
    mesh=_MESH,
    compiler_params=_CP,
    scratch_types=[
        pltpu.VMEM((K, CHUNK), jnp.int32),
        pltpu.VMEM((K, CHUNK), jnp.int32),
        pltpu.VMEM((N_PAD,), jnp.float32),
        pltpu.VMEM((N_PAD,), jnp.float32),
    ],
)
def _sc_hist(srcp_hbm, dstp_hbm, deg_hbm, sidx, didx, hsrc, hdst):
    c = lax.axis_index("c")
    s = lax.axis_index("s")
    wid = s * NC + c
    pltpu.sync_copy(srcp_hbm.at[wid], sidx)
    pltpu.sync_copy(dstp_hbm.at[wid], didx)

    @pl.loop(0, N_PAD, step=16)
    def _(i):
        z16 = jnp.zeros((16,), jnp.float32)
        hsrc[pl.ds(i, 16)] = z16
        hdst[pl.ds(i, 16)] = z16

    ones = jnp.ones((16,), jnp.float32)

    @pl.loop(0, K)
    def _(j):
        @pl.loop(0, CHUNK, step=16)
        def _(t):
            plsc.addupdate_scatter(hsrc, [sidx[j, pl.ds(t, 16)]], ones)
            plsc.addupdate_scatter(hdst, [didx[j, pl.ds(t, 16)]], ones)

    pltpu.sync_copy(hsrc, deg_hbm.at[0, wid])
    pltpu.sync_copy(hdst, deg_hbm.at[1, wid])


@functools.partial(
    pl.kernel,
    out_type=jax.ShapeDtypeStruct((NC, N_PAD, D), jnp.float32),
    mesh=_MESH,
    scratch_types=[
        pltpu.VMEM((2, G, CHUNK), jnp.int32),
        pltpu.VMEM((2, G, CHUNK), jnp.int32),
        pltpu.VMEM((2, CHUNK, D), jnp.float32),
        pltpu.VMEM_SHARED((N_PAD, D), jnp.float32),
        pltpu.SemaphoreType.DMA((2,)),
        pltpu.SemaphoreType.DMA((2,)),
        pltpu.SemaphoreType.DMA((2,)),
    ],
)
def _sc_prop(z_hbm, srcp_hbm, dstp_hbm, out_hbm, sidx, didx, gbuf,
             acc, gsem, isem, ssem):
    c = lax.axis_index("c")
    s = lax.axis_index("s")
    wid = s * NC + c
    row0 = s * ROWS_PER_SUB

    def _refill(p, j0):
        return (
            pltpu.make_async_copy(srcp_hbm.at[wid, pl.ds(j0, G)], sidx.at[p],
                                  isem.at[p]),
            pltpu.make_async_copy(dstp_hbm.at[wid, pl.ds(j0, G)], didx.at[p],
                                  isem.at[p]),
        )

    def _gather(p, r, b):
        return pltpu.make_async_copy(z_hbm.at[sidx.at[p, r]], gbuf.at[b],
                                     gsem.at[b])

    def _scatter(p, r, b):
        return pltpu.make_async_copy(gbuf.at[b], acc.at[didx.at[p, r]],
                                     ssem.at[b])

    ra, rb = _refill(0, 0)
    ra.start()
    rb.start()

    @pl.loop(0, CHUNK)
    def _(i):
        for t in range(D // 16):
            gbuf[0, i, pl.ds(t * 16, 16)] = jnp.zeros((16,), jnp.float32)

    for t in range(ROWS_PER_SUB // CHUNK):
        pltpu.sync_copy(gbuf.at[0], acc.at[pl.ds(row0 + t * CHUNK, CHUNK)])
    _ZTAIL = ROWS_PER_SUB % CHUNK
    if _ZTAIL:
        pltpu.sync_copy(
            gbuf.at[0, pl.ds(0, _ZTAIL)],
            acc.at[pl.ds(row0 + ROWS_PER_SUB - _ZTAIL, _ZTAIL)],
        )

    ra.wait()
    rb.wait()
    plsc.subcore_barrier()

    _gather(0, 0, 0).start()

    @pl.loop(0, K, step=2 * G)
    def _(j):
        for cc in range(2 * G):
            p, r, b = cc // G, cc % G, cc % 2
            pp, rp, bp = (cc - 1) // G % 2, (cc - 1) % G, (cc - 1) % 2
            pn, rn, bn = (cc + 1) // G % 2, (cc + 1) % G, (cc + 1) % 2

            if cc == G - 1:
                wa, wb = _refill(1, 0)
                wa.wait()
                wb.wait()

            if cc == 2 * G - 1:
                @pl.when(j + 2 * G < K)
                def _():
                    wa, wb = _refill(0, 0)
                    wa.wait()
                    wb.wait()

            _gather(p, r, b).wait()
            _scatter(p, r, b).start(add=True)

            if cc == 0:
                @pl.when(j > 0)
                def _():
                    _scatter(1, G - 1, 1).wait()
            else:
                _scatter(pp, rp, bp).wait()

            if cc < 2 * G - 1:
                _gather(pn, rn, bn).start()
            else:
                @pl.when(j + 2 * G < K)
                def _():
                    _gather(0, 0, bn).start()

            if cc == 1:
                wa, wb = _refill(1, j + G)
                wa.start()
                wb.start()

            if cc == G:
                @pl.when(j + 2 * G < K)
                def _():
                    wa, wb = _refill(0, j + 2 * G)
                    wa.start()
                    wb.start()

    _scatter(1, G - 1, 1).wait()
    plsc.subcore_barrier()
    pltpu.sync_copy(
        acc.at[pl.ds(row0, ROWS_PER_SUB)],
        out_hbm.at[c, pl.ds(row0, ROWS_PER_SUB)],
    )


def _tc_prep_body(deg_ref, feat_ref, nsrc_ref, ndst_ref, z_ref):
    deg = jnp.sum(deg_ref[...], axis=1)
    n = lax.rsqrt(jnp.maximum(deg, 1.0))
    nsrc = n[0][:, None]
    ndst = n[1][:, None]
    nsrc_ref[...] = nsrc
    ndst_ref[...] = ndst
    z_ref[pl.ds(0, N), :] = feat_ref[...] * nsrc[0:N]
    z_ref[pl.ds(N, N_TRASH), :] = jnp.zeros((N_TRASH, D), jnp.float32)


_tc_prep = pl.pallas_call(
    _tc_prep_body,
    out_shape=(
        jax.ShapeDtypeStruct((N_PAD, 1), jnp.float32),
        jax.ShapeDtypeStruct((N_PAD, 1), jnp.float32),
        jax.ShapeDtypeStruct((N_PAD, D), jnp.float32),
    ),
)


def _tc_layer_body(scale_out, p_ref, ndst_ref, nsrc_ref, w_ref, b_ref, o_ref):
    agg = (p_ref[0] + p_ref[1]) * ndst_ref[...]
    h = jnp.dot(agg, w_ref[...], preferred_element_type=jnp.float32) + b_ref[...]
    if scale_out:
        o_ref[...] = h * nsrc_ref[...]
    else:
        o_ref[...] = h[0:N]


_tc_mid_layer = pl.pallas_call(
    functools.partial(_tc_layer_body, True),
    out_shape=jax.ShapeDtypeStruct((N_PAD, D), jnp.float32),
)

_tc_last_layer = pl.pallas_call(
    functools.partial(_tc_layer_body, False),
    out_shape=jax.ShapeDtypeStruct((N, D), jnp.float32),
)


def kernel(in_feat, edge_index, W1, b1, W2, b2):
    src = edge_index[0].astype(jnp.int32)
    dst = edge_index[1].astype(jnp.int32)
    pad_ids = N + (jnp.arange(PAD, dtype=jnp.int32) % N_TRASH)
    srcp = jnp.concatenate([src, pad_ids]).reshape(NW, K, CHUNK)
    dstp = jnp.concatenate([dst, pad_ids]).reshape(NW, K, CHUNK)
    degs = _sc_hist(srcp, dstp)
    nsrc, ndst, z1 = _tc_prep(degs, in_feat)
    p = _sc_prop(z1, srcp, dstp)
    z2 = _tc_mid_layer(p, ndst, nsrc, W1, b1.reshape(1, D))
    q = _sc_prop(z2, srcp, dstp)
    return _tc_last_layer(q, ndst, nsrc, W2, b2.reshape(1, D))


# speedup vs baseline: 1.0048x; 1.0048x over previous
import dataclasses
import functools

import jax
import jax.numpy as jnp
from jax import lax
from jax.experimental import pallas as pl
from jax.experimental.pallas import tpu as pltpu
from jax.experimental.pallas import tpu_sc as plsc

N = 10000
E = 320000
D = 128

NC = 2
NS = 16
NW = NC * NS

CHUNK = 128
K = 80
E_PAD = NW * K * CHUNK
PAD = E_PAD - E
G = 8
N_PAD = 10112
N_TRASH = N_PAD - N
ROWS_PER_SUB = N_PAD // NS

_MESH = plsc.VectorSubcoreMesh(
    core_axis_name="c", subcore_axis_name="s", num_cores=NC, num_subcores=NS
)

_CP = pltpu.CompilerParams()
if "needs_layout_passes" in pltpu.CompilerParams.__dataclass_fields__:
    _CP = dataclasses.replace(_CP, needs_layout_passes=False)


@functools.partial(
    pl.kernel,
    out_type=jax.ShapeDtypeStruct((2, NW, N_PAD), jnp.float32),
    mesh=_MESH,
    compiler_params=_CP,
    scratch_types=[
        pltpu.VMEM((K, CHUNK), jnp.int32),
        pltpu.VMEM((K, CHUNK), jnp.int32),
        pltpu.VMEM((N_PAD,), jnp.float32),
        pltpu.VMEM((N_PAD,), jnp.float32),
    ],
)
def _sc_hist(srcp_hbm, dstp_hbm, deg_hbm, sidx, didx, hsrc, hdst):
    c = lax.axis_index("c")
    s = lax.axis_index("s")
    wid = s * NC + c
    pltpu.sync_copy(srcp_hbm.at[wid], sidx)
    pltpu.sync_copy(dstp_hbm.at[wid], didx)

    @pl.loop(0, N_PAD, step=128)
    def _(i):
        z16 = jnp.zeros((16,), jnp.float32)
        for t in range(8):
            hsrc[pl.ds(i + t * 16, 16)] = z16
            hdst[pl.ds(i + t * 16, 16)] = z16

    ones = jnp.ones((16,), jnp.float32)

    @pl.loop(0, K)
    def _(j):
        for t in range(CHUNK // 16):
            plsc.addupdate_scatter(hsrc, [sidx[j, pl.ds(t * 16, 16)]], ones)
            plsc.addupdate_scatter(hdst, [didx[j, pl.ds(t * 16, 16)]], ones)

    pltpu.sync_copy(hsrc, deg_hbm.at[0, wid])
    pltpu.sync_copy(hdst, deg_hbm.at[1, wid])


@functools.partial(
    pl.kernel,
    out_type=jax.ShapeDtypeStruct((NC, N_PAD, D), jnp.float32),
    mesh=_MESH,
    scratch_types=[
        pltpu.VMEM((2, G, CHUNK), jnp.int32),
        pltpu.VMEM((2, G, CHUNK), jnp.int32),
        pltpu.VMEM((2, CHUNK, D), jnp.float32),
        pltpu.VMEM_SHARED((N_PAD, D), jnp.float32),
        pltpu.SemaphoreType.DMA((2,)),
        pltpu.SemaphoreType.DMA((2,)),
    ],
)
def _sc_prop(z_hbm, srcp_hbm, dstp_hbm, out_hbm, sidx, didx, gbuf,
             acc, gsem, isem):
    c = lax.axis_index("c")
    s = lax.axis_index("s")
    wid = s * NC + c
    row0 = s * ROWS_PER_SUB

    def _refill(p, j0):
        return (
            pltpu.make_async_copy(srcp_hbm.at[wid, pl.ds(j0, G)], sidx.at[p],
                                  isem.at[p]),
            pltpu.make_async_copy(dstp_hbm.at[wid, pl.ds(j0, G)], didx.at[p],
                                  isem.at[p]),
        )

    def _gather(p, r, b):
        return pltpu.make_async_copy(z_hbm.at[sidx.at[p, r]], gbuf.at[b],
                                     gsem.at[b])

    def _scatter(p, r, b):
        pltpu.sync_copy(gbuf.at[b], acc.at[didx.at[p, r]], add=True)

    ra, rb = _refill(0, 0)
    ra.start()
    rb.start()
    ra2, rb2 = _refill(1, G)
    ra2.start()
    rb2.start()

    @pl.loop(0, CHUNK)
    def _(i):
        for t in range(D // 16):
            gbuf[0, i, pl.ds(t * 16, 16)] = jnp.zeros((16,), jnp.float32)

    for t in range(ROWS_PER_SUB // CHUNK):
        pltpu.sync_copy(gbuf.at[0], acc.at[pl.ds(row0 + t * CHUNK, CHUNK)])
    _ZTAIL = ROWS_PER_SUB % CHUNK
    if _ZTAIL:
        pltpu.sync_copy(
            gbuf.at[0, pl.ds(0, _ZTAIL)],
            acc.at[pl.ds(row0 + ROWS_PER_SUB - _ZTAIL, _ZTAIL)],
        )

    ra.wait()
    rb.wait()
    plsc.subcore_barrier()

    _gather(0, 0, 0).start()

    @pl.loop(0, K, step=2 * G)
    def _(j):
        for cc in range(2 * G):
            p, r, b = cc // G, cc % G, cc % 2
            pn, rn, bn = (cc + 1) // G % 2, (cc + 1) % G, (cc + 1) % 2

            if cc == G - 1:
                wa, wb = _refill(1, 0)
                wa.wait()
                wb.wait()

            if cc == 2 * G - 1:
                @pl.when(j + 2 * G < K)
                def _():
                    wa, wb = _refill(0, 0)
                    wa.wait()
                    wb.wait()

            _gather(p, r, b).wait()

            if cc < 2 * G - 1:
                _gather(pn, rn, bn).start()
            else:
                @pl.when(j + 2 * G < K)
                def _():
                    _gather(0, 0, bn).start()

            _scatter(p, r, b)

            if cc == G - 1:
                @pl.when(j + 2 * G < K)
                def _():
                    wa, wb = _refill(0, j + 2 * G)
                    wa.start()
                    wb.start()

            if cc == 2 * G - 1:
                @pl.when(j + 3 * G < K)
                def _():
                    wa, wb = _refill(1, j + 3 * G)
                    wa.start()
                    wb.start()

    plsc.subcore_barrier()
    pltpu.sync_copy(
        acc.at[pl.ds(row0, ROWS_PER_SUB)],
        out_hbm.at[c, pl.ds(row0, ROWS_PER_SUB)],
    )


def _tc_prep_body(deg_ref, feat_ref, nsrc_ref, ndst_ref, z_ref):
    deg = jnp.sum(deg_ref[...], axis=1)
    n = lax.rsqrt(jnp.maximum(deg, 1.0))
    nsrc = n[0][:, None]
    ndst = n[1][:, None]
    nsrc_ref[...] = nsrc
    ndst_ref[...] = ndst
    z_ref[pl.ds(0, N), :] = feat_ref[...] * nsrc[0:N]
    z_ref[pl.ds(N, N_TRASH), :] = jnp.zeros((N_TRASH, D), jnp.float32)


_tc_prep = pl.pallas_call(
    _tc_prep_body,
    out_shape=(
        jax.ShapeDtypeStruct((N_PAD, 1), jnp.float32),
        jax.ShapeDtypeStruct((N_PAD, 1), jnp.float32),
        jax.ShapeDtypeStruct((N_PAD, D), jnp.float32),
    ),
)


def _tc_layer_body(scale_out, p_ref, ndst_ref, nsrc_ref, w_ref, b_ref, o_ref):
    agg = (p_ref[0] + p_ref[1]) * ndst_ref[...]
    h = jnp.dot(agg, w_ref[...], preferred_element_type=jnp.float32) + b_ref[...]
    if scale_out:
        o_ref[...] = h * nsrc_ref[...]
    else:
        o_ref[...] = h[0:N]


_tc_mid_layer = pl.pallas_call(
    functools.partial(_tc_layer_body, True),
    out_shape=jax.ShapeDtypeStruct((N_PAD, D), jnp.float32),
)

_tc_last_layer = pl.pallas_call(
    functools.partial(_tc_layer_body, False),
    out_shape=jax.ShapeDtypeStruct((N, D), jnp.float32),
)


def kernel(in_feat, edge_index, W1, b1, W2, b2):
    src = edge_index[0].astype(jnp.int32)
    dst = edge_index[1].astype(jnp.int32)
    pad_ids = N + (jnp.arange(PAD, dtype=jnp.int32) % N_TRASH)
    srcp = jnp.concatenate([src, pad_ids]).reshape(NW, K, CHUNK)
    dstp = jnp.concatenate([dst, pad_ids]).reshape(NW, K, CHUNK)
    degs = _sc_hist(srcp, dstp)
    nsrc, ndst, z1 = _tc_prep(degs, in_feat)
    p = _sc_prop(z1, srcp, dstp)
    z2 = _tc_mid_layer(p, ndst, nsrc, W1, b1.reshape(1, D))
    q = _sc_prop(z2, srcp, dstp)
    return _tc_last_layer(q, ndst, nsrc, W2, b2.reshape(1, D))
